# baseline (speedup 1.0000x reference)
import functools

import jax
import jax.numpy as jnp
from jax import lax
from jax.experimental import pallas as pl
from jax.experimental.pallas import tpu as pltpu
from jax.experimental.pallas import tpu_sc as plsc

NUM_CORES = 2
NUM_SUBCORES = 16
NUM_WORKERS = NUM_CORES * NUM_SUBCORES

GATHER = 128
CHUNK_GATHERS = 8
CHUNK = GATHER * CHUNK_GATHERS


def _sc_embedding_lookup(table, idx_flat):
    B = idx_flat.shape[0]
    D = table.shape[1]
    rows_per_w = B // NUM_WORKERS
    chunks_per_w = rows_per_w // CHUNK
    assert rows_per_w % CHUNK == 0 and chunks_per_w % 2 == 0
    npairs = chunks_per_w // 2

    mesh = plsc.VectorSubcoreMesh(core_axis_name="c", subcore_axis_name="s")

    @functools.partial(
        pl.kernel,
        out_type=jax.ShapeDtypeStruct((B, D), jnp.float32),
        mesh=mesh,
        compiler_params=pltpu.CompilerParams(use_tc_tiling_on_sc=False),
        scratch_types=[
            pltpu.VMEM((2, CHUNK), jnp.int32),
            pltpu.VMEM((2, CHUNK, D), jnp.float32),
            pltpu.SemaphoreType.DMA,
            pltpu.SemaphoreType.DMA,
            pltpu.SemaphoreType.DMA,
            pltpu.SemaphoreType.DMA,
        ],
    )
    def k(table_hbm, idx_hbm, out_hbm, idx_v, rows_v, gsem0, gsem1, ssem0, ssem1):
        wid = lax.axis_index("s") * NUM_CORES + lax.axis_index("c")
        row0 = wid * rows_per_w

        def idx_load(c, slot):
            base = pl.multiple_of(row0 + c * CHUNK, CHUNK)
            pltpu.sync_copy(idx_hbm.at[pl.ds(base, CHUNK)], idx_v.at[slot])

        def fire_gathers(c, slot, sem):
            del c
            pltpu.async_copy(
                table_hbm.at[idx_v.at[slot]], rows_v.at[slot], sem
            )

        def drain_gathers(slot, sem):
            pltpu.make_async_copy(
                table_hbm.at[pl.ds(0, CHUNK)], rows_v.at[slot], sem
            ).wait()

        def fire_store(c, slot, sem):
            base = pl.multiple_of(row0 + c * CHUNK, CHUNK)
            pltpu.async_copy(rows_v.at[slot], out_hbm.at[pl.ds(base, CHUNK)], sem)

        def drain_store(slot, sem):
            pltpu.make_async_copy(
                rows_v.at[slot], out_hbm.at[pl.ds(0, CHUNK)], sem
            ).wait()

        idx_load(0, 0)
        fire_gathers(0, 0, gsem0)

        def pair(gi, carry):
            c0 = 2 * gi
            idx_load(c0 + 1, 1)
            drain_gathers(0, gsem0)

            @pl.when(gi >= 1)
            def _():
                drain_store(1, ssem1)

            fire_gathers(c0 + 1, 1, gsem1)
            fire_store(c0, 0, ssem0)

            drain_gathers(1, gsem1)
            drain_store(0, ssem0)

            @pl.when(gi < npairs - 1)
            def _():
                idx_load(c0 + 2, 0)
                fire_gathers(c0 + 2, 0, gsem0)

            fire_store(c0 + 1, 1, ssem1)
            return carry

        lax.fori_loop(0, npairs, pair, 0)
        drain_store(1, ssem1)

    return k(table, idx_flat)


def kernel(indices, weight):
    B = indices.shape[0] * indices.shape[1]
    idx_flat = indices.reshape(B).astype(jnp.int32)
    out = _sc_embedding_lookup(weight, idx_flat)
    return out.reshape(indices.shape + (weight.shape[1],))
